# Initial kernel scaffold; baseline (speedup 1.0000x reference)
import jax
import jax.numpy as jnp
from jax.experimental import pallas as pl


def kernel(x, x_lengths, w_ih_t_0, w_hh_t_0, bias_0, w_ih_t_1, w_hh_t_1, bias_1, w_ih_t_2, w_hh_t_2, bias_2, w_fc_t, b_fc):
    raise NotImplementedError("write your pallas kernel here")


# speedup vs baseline: 3.9029x; 3.9029x over previous
import jax
import jax.numpy as jnp
from jax.experimental import pallas as pl
from jax.experimental.pallas import tpu as pltpu


def _make_body(T, H):
    def body(x_ref, len_ref,
             w_ih0_ref, w_hh0_ref,
             w_ih1_ref, w_hh1_ref,
             w_ih2_ref, w_hh2_ref,
             b0_ref, b1_ref, b2_ref,
             w_fc_ref, b_fc_ref,
             out_ref, xp_scr):
        BT = out_ref.shape[0]

        x2d = x_ref[...].reshape(T * BT, -1)
        xp = jnp.dot(x2d, w_ih0_ref[...], preferred_element_type=jnp.float32)
        xp_scr[...] = (xp + b0_ref[...]).reshape(T, BT, 4 * H)

        w_hh0 = w_hh0_ref[...]
        w_ih1 = w_ih1_ref[...]
        w_hh1 = w_hh1_ref[...]
        w_ih2 = w_ih2_ref[...]
        w_hh2 = w_hh2_ref[...]
        b1 = jnp.broadcast_to(b1_ref[...], (BT, 4 * H))
        b2 = jnp.broadcast_to(b2_ref[...], (BT, 4 * H))
        lens = len_ref[...]

        def cell(gates, c_prev, h_prev_bf, mask):
            i_f = jax.nn.sigmoid(gates[:, :2 * H])
            i_g = i_f[:, :H]
            f_g = i_f[:, H:]
            g_g = jnp.tanh(gates[:, 2 * H:3 * H])
            o_g = jax.nn.sigmoid(gates[:, 3 * H:])
            c_new = f_g * c_prev + i_g * g_g
            h_new = o_g * jnp.tanh(c_new)
            c_upd = jnp.where(mask, c_new, c_prev)
            h_upd = jnp.where(mask, h_new.astype(jnp.bfloat16), h_prev_bf)
            return h_upd, c_upd

        def step(t, carry):
            h0, h1, h2, c0, c1, c2 = carry
            mask = lens > t
            g0 = xp_scr[t] + jnp.dot(h0, w_hh0,
                                     preferred_element_type=jnp.float32)
            h0, c0 = cell(g0, c0, h0, mask)
            g1 = (jnp.dot(h0, w_ih1, preferred_element_type=jnp.float32)
                  + jnp.dot(h1, w_hh1, preferred_element_type=jnp.float32)
                  + b1)
            h1, c1 = cell(g1, c1, h1, mask)
            g2 = (jnp.dot(h1, w_ih2, preferred_element_type=jnp.float32)
                  + jnp.dot(h2, w_hh2, preferred_element_type=jnp.float32)
                  + b2)
            h2, c2 = cell(g2, c2, h2, mask)
            return h0, h1, h2, c0, c1, c2

        hz = jnp.zeros((BT, H), jnp.bfloat16)
        cz = jnp.zeros((BT, H), jnp.float32)
        h0, h1, h2, c0, c1, c2 = jax.lax.fori_loop(
            0, T, step, (hz, hz, hz, cz, cz, cz), unroll=True)

        out_ref[...] = (jnp.dot(h2, w_fc_ref[...],
                                preferred_element_type=jnp.float32)
                        + b_fc_ref[...]).astype(out_ref.dtype)

    return body


def kernel(x, x_lengths, w_ih_t_0, w_hh_t_0, bias_0, w_ih_t_1, w_hh_t_1,
           bias_1, w_ih_t_2, w_hh_t_2, bias_2, w_fc_t, b_fc):
    B, T, D = x.shape
    H = w_hh_t_0.shape[0]
    O = w_fc_t.shape[1]
    BT = 128
    assert B % BT == 0

    cdt = jnp.bfloat16

    x_tm = jnp.transpose(x, (1, 0, 2)).astype(cdt)
    lens = x_lengths.astype(jnp.int32).reshape(B, 1)

    weights = [w_ih_t_0.astype(cdt), w_hh_t_0.astype(cdt),
               w_ih_t_1.astype(cdt), w_hh_t_1.astype(cdt),
               w_ih_t_2.astype(cdt), w_hh_t_2.astype(cdt)]
    biases = [bias_0, bias_1, bias_2]
    w_fc = w_fc_t.astype(cdt)

    def resident(a):
        n = a.ndim
        return pl.BlockSpec(a.shape, lambda b, _n=n: (0,) * _n)

    in_specs = [
        pl.BlockSpec((T, BT, D), lambda b: (0, b, 0)),
        pl.BlockSpec((BT, 1), lambda b: (b, 0)),
        *[resident(w) for w in weights],
        *[resident(b) for b in biases],
        resident(w_fc),
        resident(b_fc),
    ]

    return pl.pallas_call(
        _make_body(T, H),
        out_shape=jax.ShapeDtypeStruct((B, O), jnp.float32),
        grid=(B // BT,),
        in_specs=in_specs,
        out_specs=pl.BlockSpec((BT, O), lambda b: (b, 0)),
        scratch_shapes=[pltpu.VMEM((T, BT, 4 * H), jnp.float32)],
        compiler_params=pltpu.CompilerParams(
            dimension_semantics=("parallel",),
            vmem_limit_bytes=100 * 1024 * 1024,
        ),
    )(x_tm, lens, *weights, *biases, w_fc, b_fc)
